# baseline (speedup 1.0000x reference)
import functools
import jax
import jax.numpy as jnp
from jax import lax
from jax.experimental import pallas as pl
from jax.experimental.pallas import tpu as pltpu
from jax.experimental.pallas import tpu_sc as plsc

_NC = 2
_NS = 16
_L = 16

_C = 32
_CHUNKS0 = 5
_CHUNKS1 = 3


def _pipeline(x_hbm, pe_hbm, o_hbm, pe_bufs, x_bufs, s_pe, s_in, s_out,
              s0, n_chunks):
    b, s, d = x_hbm.shape
    units = [(c, bi) for c in range(n_chunks) for bi in range(b)]
    n_u = len(units)

    pe_h, in_h, out_h = {}, {}, {}
    pe_h[0] = pltpu.async_copy(pe_hbm.at[pl.ds(s0, _C)], pe_bufs[0], s_pe[0])
    in_h[0] = pltpu.async_copy(x_hbm.at[0, pl.ds(s0, _C)], x_bufs[0], s_in[0])

    for u in range(n_u):
        c, bi = units[u]
        buf = x_bufs[u % 2]
        if u + 1 < n_u:
            c2, bi2 = units[u + 1]
            if bi2 == 0:
                pe_h[c2] = pltpu.async_copy(
                    pe_hbm.at[pl.ds(s0 + c2 * _C, _C)], pe_bufs[c2 % 2],
                    s_pe[c2 % 2])
            if u >= 1:
                out_h[u - 1].wait()
            in_h[u + 1] = pltpu.async_copy(
                x_hbm.at[bi2, pl.ds(s0 + c2 * _C, _C)], x_bufs[(u + 1) % 2],
                s_in[(u + 1) % 2])
        if bi == 0:
            pe_h[c].wait()
        in_h[u].wait()
        pe_buf = pe_bufs[c % 2]

        def add_row(r, carry, buf=buf, pe_buf=pe_buf):
            for j in range(d // _L):
                sl = pl.ds(j * _L, _L)
                buf[r, sl] = buf[r, sl] + pe_buf[r, sl]
            return carry

        lax.fori_loop(0, _C, add_row, 0)
        out_h[u] = pltpu.async_copy(
            buf, o_hbm.at[bi, pl.ds(s0 + c * _C, _C)], s_out[u % 2])

    out_h[n_u - 2].wait()
    out_h[n_u - 1].wait()


def _sc_body(x_hbm, pe_hbm, o_hbm,
             pe0, pe1, xb0, xb1,
             s_pe0, s_pe1, s_in0, s_in1, s_out0, s_out1):
    cid = lax.axis_index("c")
    sid = lax.axis_index("s")
    pe_bufs, x_bufs = [pe0, pe1], [xb0, xb1]
    s_pe, s_in, s_out = [s_pe0, s_pe1], [s_in0, s_in1], [s_out0, s_out1]

    @pl.when(cid == 0)
    def _():
        _pipeline(x_hbm, pe_hbm, o_hbm, pe_bufs, x_bufs, s_pe, s_in, s_out,
                  sid * (_CHUNKS0 * _C), _CHUNKS0)

    @pl.when(cid == 1)
    def _():
        _pipeline(x_hbm, pe_hbm, o_hbm, pe_bufs, x_bufs, s_pe, s_in, s_out,
                  _NS * _CHUNKS0 * _C + sid * (_CHUNKS1 * _C), _CHUNKS1)


def kernel(x, pos_emb):
    b, s, d = x.shape
    pe = pos_emb[:s]
    mesh = plsc.VectorSubcoreMesh(core_axis_name="c", subcore_axis_name="s")
    k = functools.partial(
        pl.kernel,
        mesh=mesh,
        out_type=jax.ShapeDtypeStruct((b, s, d), x.dtype),
        scratch_types=(
            [pltpu.VMEM((_C, d), jnp.float32)] * 4
            + [pltpu.SemaphoreType.DMA] * 6
        ),
    )(_sc_body)
    return k(x, pe)


# speedup vs baseline: 1.0065x; 1.0065x over previous
import functools
import jax
import jax.numpy as jnp
from jax import lax
from jax.experimental import pallas as pl
from jax.experimental.pallas import tpu as pltpu
from jax.experimental.pallas import tpu_sc as plsc

_NC = 2
_NS = 16
_L = 16

_C = 32
_CHUNKS0 = 3
_CHUNKS1 = 5


def _pipeline(x_hbm, pe_hbm, o_hbm, pe_bufs, x_bufs, s_pe, s_in, s_out,
              s0, n_chunks):
    b, s, d = x_hbm.shape
    units = [(c, bi) for c in range(n_chunks) for bi in range(b)]
    n_u = len(units)

    pe_h, in_h, out_h = {}, {}, {}
    pe_h[0] = pltpu.async_copy(pe_hbm.at[pl.ds(s0, _C)], pe_bufs[0], s_pe[0])
    in_h[0] = pltpu.async_copy(x_hbm.at[0, pl.ds(s0, _C)], x_bufs[0], s_in[0])

    for u in range(n_u):
        c, bi = units[u]
        buf = x_bufs[u % 2]
        if u + 1 < n_u:
            c2, bi2 = units[u + 1]
            if bi2 == 0:
                pe_h[c2] = pltpu.async_copy(
                    pe_hbm.at[pl.ds(s0 + c2 * _C, _C)], pe_bufs[c2 % 2],
                    s_pe[c2 % 2])
            if u >= 1:
                out_h[u - 1].wait()
            in_h[u + 1] = pltpu.async_copy(
                x_hbm.at[bi2, pl.ds(s0 + c2 * _C, _C)], x_bufs[(u + 1) % 2],
                s_in[(u + 1) % 2])
        if bi == 0:
            pe_h[c].wait()
        in_h[u].wait()
        pe_buf = pe_bufs[c % 2]

        def add_row(r, carry, buf=buf, pe_buf=pe_buf):
            for j in range(d // _L):
                sl = pl.ds(j * _L, _L)
                buf[r, sl] = buf[r, sl] + pe_buf[r, sl]
            return carry

        lax.fori_loop(0, _C, add_row, 0)
        out_h[u] = pltpu.async_copy(
            buf, o_hbm.at[bi, pl.ds(s0 + c * _C, _C)], s_out[u % 2])

    out_h[n_u - 2].wait()
    out_h[n_u - 1].wait()


def _sc_body(x_hbm, pe_hbm, o_hbm,
             pe0, pe1, xb0, xb1,
             s_pe0, s_pe1, s_in0, s_in1, s_out0, s_out1):
    cid = lax.axis_index("c")
    sid = lax.axis_index("s")
    pe_bufs, x_bufs = [pe0, pe1], [xb0, xb1]
    s_pe, s_in, s_out = [s_pe0, s_pe1], [s_in0, s_in1], [s_out0, s_out1]

    @pl.when(cid == 0)
    def _():
        _pipeline(x_hbm, pe_hbm, o_hbm, pe_bufs, x_bufs, s_pe, s_in, s_out,
                  sid * (_CHUNKS0 * _C), _CHUNKS0)

    @pl.when(cid == 1)
    def _():
        _pipeline(x_hbm, pe_hbm, o_hbm, pe_bufs, x_bufs, s_pe, s_in, s_out,
                  _NS * _CHUNKS0 * _C + sid * (_CHUNKS1 * _C), _CHUNKS1)


def kernel(x, pos_emb):
    b, s, d = x.shape
    pe = pos_emb[:s]
    mesh = plsc.VectorSubcoreMesh(core_axis_name="c", subcore_axis_name="s")
    k = functools.partial(
        pl.kernel,
        mesh=mesh,
        out_type=jax.ShapeDtypeStruct((b, s, d), x.dtype),
        scratch_types=(
            [pltpu.VMEM((_C, d), jnp.float32)] * 4
            + [pltpu.SemaphoreType.DMA] * 6
        ),
    )(_sc_body)
    return k(x, pe)


# speedup vs baseline: 1.2089x; 1.2011x over previous
import functools
import jax
import jax.numpy as jnp
from jax import lax
from jax.experimental import pallas as pl
from jax.experimental.pallas import tpu as pltpu
from jax.experimental.pallas import tpu_sc as plsc

_NC = 2
_NS = 16
_NW = _NC * _NS
_L = 16

_C = 32


def _sc_body(x_hbm, pe_hbm, o_hbm,
             pe0, pe1, xb0, xb1,
             s_pe0, s_pe1, s_in0, s_in1, s_out0, s_out1):
    b, s, d = x_hbm.shape
    rows_per_w = s // _NW
    n_chunks = rows_per_w // _C
    wid = lax.axis_index("s") * _NC + lax.axis_index("c")
    s0 = wid * rows_per_w

    pe_bufs, x_bufs = [pe0, pe1], [xb0, xb1]
    s_pe, s_in, s_out = [s_pe0, s_pe1], [s_in0, s_in1], [s_out0, s_out1]
    units = [(c, bi) for c in range(n_chunks) for bi in range(b)]
    n_u = len(units)

    pe_h, in_h, out_h = {}, {}, {}
    pe_h[0] = pltpu.async_copy(pe_hbm.at[pl.ds(s0, _C)], pe_bufs[0], s_pe[0])
    in_h[0] = pltpu.async_copy(x_hbm.at[0, pl.ds(s0, _C)], x_bufs[0], s_in[0])

    for u in range(n_u):
        c, bi = units[u]
        buf = x_bufs[u % 2]
        if u + 1 < n_u:
            c2, bi2 = units[u + 1]
            if bi2 == 0:
                pe_h[c2] = pltpu.async_copy(
                    pe_hbm.at[pl.ds(s0 + c2 * _C, _C)], pe_bufs[c2 % 2],
                    s_pe[c2 % 2])
            if u >= 1:
                out_h[u - 1].wait()
            in_h[u + 1] = pltpu.async_copy(
                x_hbm.at[bi2, pl.ds(s0 + c2 * _C, _C)], x_bufs[(u + 1) % 2],
                s_in[(u + 1) % 2])
        if bi == 0:
            pe_h[c].wait()
        in_h[u].wait()
        pe_buf = pe_bufs[c % 2]

        def add_row(r, carry, buf=buf, pe_buf=pe_buf):
            for j in range(d // _L):
                sl = pl.ds(j * _L, _L)
                buf[r, sl] = buf[r, sl] + pe_buf[r, sl]
            return carry

        lax.fori_loop(0, _C, add_row, 0)
        out_h[u] = pltpu.async_copy(
            buf, o_hbm.at[bi, pl.ds(s0 + c * _C, _C)], s_out[u % 2])

    out_h[n_u - 2].wait()
    out_h[n_u - 1].wait()


def kernel(x, pos_emb):
    b, s, d = x.shape
    pe = pos_emb[:s]
    mesh = plsc.VectorSubcoreMesh(core_axis_name="c", subcore_axis_name="s")
    k = functools.partial(
        pl.kernel,
        mesh=mesh,
        out_type=jax.ShapeDtypeStruct((b, s, d), x.dtype),
        scratch_types=(
            [pltpu.VMEM((_C, d), jnp.float32)] * 4
            + [pltpu.SemaphoreType.DMA] * 6
        ),
    )(_sc_body)
    return k(x, pe)
